# baseline (speedup 1.0000x reference)
import functools

import jax
import jax.numpy as jnp
from jax import lax
from jax.experimental import pallas as pl
from jax.experimental.pallas import tpu as pltpu
from jax.experimental.pallas import tpu_sc as plsc

NC = 2
NS = 16
LANES = 16
CHUNK = 128
TPC = 3 * CHUNK


@functools.cache
def _build(B, N, D):
    NW = NC * NS
    b_per_w = B // NW
    n_chunks = b_per_w // CHUNK
    groups_per_chunk = CHUNK // LANES

    mesh = plsc.VectorSubcoreMesh(
        core_axis_name="c", subcore_axis_name="s",
        num_cores=NC, num_subcores=NS,
    )

    @functools.partial(
        pl.kernel,
        out_type=jax.ShapeDtypeStruct((B,), jnp.float32),
        mesh=mesh,
        compiler_params=pltpu.CompilerParams(
            needs_layout_passes=False, use_tc_tiling_on_sc=True),
        scratch_types=[
            pltpu.VMEM((3 * b_per_w,), jnp.int32),
            pltpu.VMEM((2 * TPC, D), jnp.float32),
            pltpu.VMEM((b_per_w,), jnp.float32),
            pltpu.SemaphoreType.DMA((2,)),
        ],
    )
    def kern(ijk_hbm, u_hbm, v_hbm, out_hbm, idxv, rows, outv, sem):
        wid = lax.axis_index("s") * NC + lax.axis_index("c")
        base = wid * b_per_w

        pltpu.sync_copy(ijk_hbm.at[pl.ds(base, b_per_w)],
                        idxv.at[pl.ds(0, b_per_w)])
        pltpu.sync_copy(ijk_hbm.at[pl.ds(B + base, b_per_w)],
                        idxv.at[pl.ds(b_per_w, b_per_w)])
        pltpu.sync_copy(ijk_hbm.at[pl.ds(2 * B + base, b_per_w)],
                        idxv.at[pl.ds(2 * b_per_w, b_per_w)])

        def issue_chunk(c, slot):
            def issue_group(g, _):
                off = c * CHUNK + g * LANES
                ivec = idxv[pl.ds(off, LANES)]
                jvec = idxv[pl.ds(b_per_w + off, LANES)]
                kvec = idxv[pl.ds(2 * b_per_w + off, LANES)]
                rbase = slot * TPC + g * LANES
                for l in range(LANES):
                    pltpu.async_copy(
                        u_hbm.at[ivec[l]], rows.at[rbase + l], sem.at[slot])
                    pltpu.async_copy(
                        v_hbm.at[jvec[l]], rows.at[CHUNK + rbase + l],
                        sem.at[slot])
                    pltpu.async_copy(
                        v_hbm.at[kvec[l]], rows.at[2 * CHUNK + rbase + l],
                        sem.at[slot])
                return _

            lax.fori_loop(0, groups_per_chunk, issue_group, None)

        def drain_slot(slot):
            sl = pl.ds(slot * TPC, TPC)
            pltpu.make_async_copy(u_hbm.at[pl.ds(0, TPC)], rows.at[sl],
                                  sem.at[slot]).wait()

        lane = lax.iota(jnp.int32, LANES)

        def compute_chunk(c, slot):
            def group_body(g, _):
                rid = lane + (slot * TPC + g * LANES)

                def d_body(d8, acc):
                    for dd in range(8):
                        col = jnp.full((LANES,), dd, jnp.int32) + d8 * 8
                        u_d = plsc.load_gather(rows, [rid, col])
                        vj_d = plsc.load_gather(rows, [rid + CHUNK, col])
                        vk_d = plsc.load_gather(rows, [rid + 2 * CHUNK, col])
                        dj = u_d - vj_d
                        dk = u_d - vk_d
                        acc = acc + (dk * dk - dj * dj)
                    return acc

                acc = lax.fori_loop(0, D // 8, d_body,
                                    jnp.zeros((LANES,), jnp.float32))
                outv[pl.ds(c * CHUNK + g * LANES, LANES)] = (
                    1.0 / (1.0 + jnp.exp(-acc)))
                return _

            lax.fori_loop(0, groups_per_chunk, group_body, None)

        issue_chunk(0, 0)
        for c in range(n_chunks):
            if c + 1 < n_chunks:
                issue_chunk(c + 1, (c + 1) % 2)
            drain_slot(c % 2)
            compute_chunk(c, c % 2)

        pltpu.sync_copy(outv, out_hbm.at[pl.ds(base, b_per_w)])

    return kern


def kernel(i, j, k, u_weight, v_weight):
    B = i.shape[0]
    N, D = u_weight.shape
    kern = _build(B, N, D)
    ijk = jnp.concatenate(
        [i.astype(jnp.int32), j.astype(jnp.int32), k.astype(jnp.int32)])
    return kern(ijk, u_weight, v_weight)


# speedup vs baseline: 1.0015x; 1.0015x over previous
import functools

import jax
import jax.numpy as jnp
from jax import lax
from jax.experimental import pallas as pl
from jax.experimental.pallas import tpu as pltpu
from jax.experimental.pallas import tpu_sc as plsc

NC = 2
NS = 16
LANES = 16
CHUNK = 128
TPC = 3 * CHUNK


@functools.cache
def _build(B, N, D):
    NW = NC * NS
    b_per_w = B // NW
    n_chunks = b_per_w // CHUNK
    groups_per_chunk = CHUNK // LANES

    mesh = plsc.VectorSubcoreMesh(
        core_axis_name="c", subcore_axis_name="s",
        num_cores=NC, num_subcores=NS,
    )

    @functools.partial(
        pl.kernel,
        out_type=jax.ShapeDtypeStruct((B,), jnp.float32),
        mesh=mesh,
        compiler_params=pltpu.CompilerParams(
            needs_layout_passes=False, use_tc_tiling_on_sc=True),
        scratch_types=[
            pltpu.VMEM((3 * b_per_w,), jnp.int32),
            pltpu.VMEM((2 * TPC, D), jnp.float32),
            pltpu.VMEM((b_per_w,), jnp.float32),
            pltpu.SemaphoreType.DMA((2,)),
        ],
    )
    def kern(ijk_hbm, u_hbm, v_hbm, out_hbm, idxv, rows, outv, sem):
        wid = lax.axis_index("s") * NC + lax.axis_index("c")
        base = wid * b_per_w

        pltpu.sync_copy(ijk_hbm.at[pl.ds(base, b_per_w)],
                        idxv.at[pl.ds(0, b_per_w)])
        pltpu.sync_copy(ijk_hbm.at[pl.ds(B + base, b_per_w)],
                        idxv.at[pl.ds(b_per_w, b_per_w)])
        pltpu.sync_copy(ijk_hbm.at[pl.ds(2 * B + base, b_per_w)],
                        idxv.at[pl.ds(2 * b_per_w, b_per_w)])

        def issue_chunk(c, slot):
            def issue_group(g, _):
                off = c * CHUNK + g * LANES
                ivec = idxv[pl.ds(off, LANES)]
                jvec = idxv[pl.ds(b_per_w + off, LANES)]
                kvec = idxv[pl.ds(2 * b_per_w + off, LANES)]
                rbase = slot * TPC + g * LANES
                for l in range(LANES):
                    pltpu.async_copy(
                        u_hbm.at[ivec[l]], rows.at[rbase + l], sem.at[slot])
                    pltpu.async_copy(
                        v_hbm.at[jvec[l]], rows.at[CHUNK + rbase + l],
                        sem.at[slot])
                    pltpu.async_copy(
                        v_hbm.at[kvec[l]], rows.at[2 * CHUNK + rbase + l],
                        sem.at[slot])
                return _

            lax.fori_loop(0, groups_per_chunk, issue_group, None)

        def drain_slot(slot):
            sl = pl.ds(slot * TPC, TPC)
            pltpu.make_async_copy(u_hbm.at[pl.ds(0, TPC)], rows.at[sl],
                                  sem.at[slot]).wait()

        lane = lax.iota(jnp.int32, LANES)

        def compute_chunk(c, slot):
            def group_body(g, _):
                rid = lane + (slot * TPC + g * LANES)

                def d_body(d8, acc):
                    for dd in range(8):
                        col = jnp.full((LANES,), dd, jnp.int32) + d8 * 8
                        u_d = plsc.load_gather(rows, [rid, col])
                        vj_d = plsc.load_gather(rows, [rid + CHUNK, col])
                        vk_d = plsc.load_gather(rows, [rid + 2 * CHUNK, col])
                        dj = u_d - vj_d
                        dk = u_d - vk_d
                        acc = acc + (dk * dk - dj * dj)
                    return acc

                acc = lax.fori_loop(0, D // 8, d_body,
                                    jnp.zeros((LANES,), jnp.float32))
                outv[pl.ds(c * CHUNK + g * LANES, LANES)] = (
                    1.0 / (1.0 + jnp.exp(-acc)))
                return _

            lax.fori_loop(0, groups_per_chunk, group_body, None)

        issue_chunk(0, 0)

        def chunk_body(c, _):
            issue_chunk(c + 1, (c + 1) % 2)
            drain_slot(c % 2)
            compute_chunk(c, c % 2)
            return _

        lax.fori_loop(0, n_chunks - 1, chunk_body, None)
        drain_slot((n_chunks - 1) % 2)
        compute_chunk(n_chunks - 1, (n_chunks - 1) % 2)

        pltpu.sync_copy(outv, out_hbm.at[pl.ds(base, b_per_w)])

    return kern


def kernel(i, j, k, u_weight, v_weight):
    B = i.shape[0]
    N, D = u_weight.shape
    kern = _build(B, N, D)
    ijk = jnp.concatenate(
        [i.astype(jnp.int32), j.astype(jnp.int32), k.astype(jnp.int32)])
    return kern(ijk, u_weight, v_weight)
